# baseline (speedup 1.0000x reference)
import jax
import jax.numpy as jnp
from jax.experimental import pallas as pl
from jax.experimental.pallas import tpu as pltpu

_SIGMA = 0.1
_NCORES = 2
_BT = 256


def _round_up(v, m):
    return ((v + m - 1) // m) * m


def _noise_linear_kernel(x_ref, w_ref, nw_ref, b_ref, nb_ref, o_ref,
                         weff_ref, beff_ref):
    j = pl.program_id(1)

    @pl.when(j == 0)
    def _():
        weff_ref[...] = (w_ref[...] + _SIGMA * nw_ref[...]).astype(jnp.bfloat16)
        beff_ref[...] = b_ref[...] + _SIGMA * nb_ref[...]

    xb = x_ref[...].astype(jnp.bfloat16)
    o_ref[...] = (
        jnp.dot(xb, weff_ref[...], preferred_element_type=jnp.float32)
        + beff_ref[...]
    )


def kernel(x, w_t, bias2d, noise_w_t, noise_b2d):
    B, K = x.shape
    Kw, N = w_t.shape
    assert K == Kw

    Bp = _round_up(B, _BT * _NCORES)
    x_p = x if Bp == B else jnp.pad(x, ((0, Bp - B), (0, 0)))
    nj = Bp // (_BT * _NCORES)

    out = pl.pallas_call(
        _noise_linear_kernel,
        grid=(_NCORES, nj),
        in_specs=[
            pl.BlockSpec((_BT, K), lambda i, j: (i * nj + j, 0)),
            pl.BlockSpec((K, N), lambda i, j: (0, 0)),
            pl.BlockSpec((K, N), lambda i, j: (0, 0)),
            pl.BlockSpec((1, N), lambda i, j: (0, 0)),
            pl.BlockSpec((1, N), lambda i, j: (0, 0)),
        ],
        out_specs=pl.BlockSpec((_BT, N), lambda i, j: (i * nj + j, 0)),
        out_shape=jax.ShapeDtypeStruct((Bp, N), jnp.float32),
        scratch_shapes=[
            pltpu.VMEM((K, N), jnp.bfloat16),
            pltpu.VMEM((1, N), jnp.float32),
        ],
        compiler_params=pltpu.CompilerParams(
            dimension_semantics=("parallel", "arbitrary"),
            vmem_limit_bytes=32 << 20,
        ),
    )(x_p, w_t, noise_w_t, bias2d, noise_b2d)

    return out if Bp == B else out[:B]


# speedup vs baseline: 1.2679x; 1.2679x over previous
import jax
import jax.numpy as jnp
from jax.experimental import pallas as pl
from jax.experimental.pallas import tpu as pltpu

_SIGMA = 0.1
_NCORES = 2
_BT = 1024


def _round_up(v, m):
    return ((v + m - 1) // m) * m


def _noise_linear_kernel(x_ref, w_ref, nw_ref, b_ref, nb_ref, o_ref,
                         weff_ref, beff_ref):
    j = pl.program_id(1)

    @pl.when(j == 0)
    def _():
        weff_ref[...] = (w_ref[...] + _SIGMA * nw_ref[...]).astype(jnp.bfloat16)
        beff_ref[...] = b_ref[...] + _SIGMA * nb_ref[...]

    xb = x_ref[...].astype(jnp.bfloat16)
    o_ref[...] = (
        jnp.dot(xb, weff_ref[...], preferred_element_type=jnp.float32)
        + beff_ref[...]
    )


def kernel(x, w_t, bias2d, noise_w_t, noise_b2d):
    B, K = x.shape
    Kw, N = w_t.shape
    assert K == Kw

    Bp = _round_up(B, _BT * _NCORES)
    x_p = x if Bp == B else jnp.pad(x, ((0, Bp - B), (0, 0)))
    nj = Bp // (_BT * _NCORES)

    out = pl.pallas_call(
        _noise_linear_kernel,
        grid=(_NCORES, nj),
        in_specs=[
            pl.BlockSpec((_BT, K), lambda i, j: (i * nj + j, 0)),
            pl.BlockSpec((K, N), lambda i, j: (0, 0)),
            pl.BlockSpec((K, N), lambda i, j: (0, 0)),
            pl.BlockSpec((1, N), lambda i, j: (0, 0)),
            pl.BlockSpec((1, N), lambda i, j: (0, 0)),
        ],
        out_specs=pl.BlockSpec((_BT, N), lambda i, j: (i * nj + j, 0)),
        out_shape=jax.ShapeDtypeStruct((Bp, N), jnp.float32),
        scratch_shapes=[
            pltpu.VMEM((K, N), jnp.bfloat16),
            pltpu.VMEM((1, N), jnp.float32),
        ],
        compiler_params=pltpu.CompilerParams(
            dimension_semantics=("parallel", "arbitrary"),
            vmem_limit_bytes=32 << 20,
        ),
    )(x_p, w_t, noise_w_t, bias2d, noise_b2d)

    return out if Bp == B else out[:B]


# speedup vs baseline: 1.3212x; 1.0420x over previous
import jax
import jax.numpy as jnp
from jax.experimental import pallas as pl
from jax.experimental.pallas import tpu as pltpu

_SIGMA = 0.1
_NCORES = 2
_BT = 2048


def _round_up(v, m):
    return ((v + m - 1) // m) * m


def _noise_linear_kernel(x_ref, w_ref, nw_ref, b_ref, nb_ref, o_ref,
                         weff_ref, beff_ref):
    j = pl.program_id(1)

    @pl.when(j == 0)
    def _():
        weff_ref[...] = (w_ref[...] + _SIGMA * nw_ref[...]).astype(jnp.bfloat16)
        beff_ref[...] = b_ref[...] + _SIGMA * nb_ref[...]

    xb = x_ref[...].astype(jnp.bfloat16)
    o_ref[...] = (
        jnp.dot(xb, weff_ref[...], preferred_element_type=jnp.float32)
        + beff_ref[...]
    )


def kernel(x, w_t, bias2d, noise_w_t, noise_b2d):
    B, K = x.shape
    Kw, N = w_t.shape
    assert K == Kw

    Bp = _round_up(B, _BT * _NCORES)
    x_p = x if Bp == B else jnp.pad(x, ((0, Bp - B), (0, 0)))
    nj = Bp // (_BT * _NCORES)

    out = pl.pallas_call(
        _noise_linear_kernel,
        grid=(_NCORES, nj),
        in_specs=[
            pl.BlockSpec((_BT, K), lambda i, j: (i * nj + j, 0)),
            pl.BlockSpec((K, N), lambda i, j: (0, 0)),
            pl.BlockSpec((K, N), lambda i, j: (0, 0)),
            pl.BlockSpec((1, N), lambda i, j: (0, 0)),
            pl.BlockSpec((1, N), lambda i, j: (0, 0)),
        ],
        out_specs=pl.BlockSpec((_BT, N), lambda i, j: (i * nj + j, 0)),
        out_shape=jax.ShapeDtypeStruct((Bp, N), jnp.float32),
        scratch_shapes=[
            pltpu.VMEM((K, N), jnp.bfloat16),
            pltpu.VMEM((1, N), jnp.float32),
        ],
        compiler_params=pltpu.CompilerParams(
            dimension_semantics=("parallel", "arbitrary"),
            vmem_limit_bytes=48 << 20,
        ),
    )(x_p, w_t, noise_w_t, bias2d, noise_b2d)

    return out if Bp == B else out[:B]
